# baseline (speedup 1.0000x reference)
import jax
import jax.numpy as jnp
from jax.experimental import pallas as pl
from jax.experimental.pallas import tpu as pltpu

M = 268
FEAT = 128
EMB = 64
TIME_SLOT = 4
GEO_THR = 3.0
T = 4 * TIME_SLOT
NH = 33
G = 10
C = 330 // G


def _gallat_kernel(day_ref, hour_ref, feat_ref, feat1_ref, fo_ref, graph_ref,
                   W_ref, af_ref, ab_ref, ag_ref, Wt_ref, Po_ref, Pd_ref,
                   tr_ref, hist_blk_ref, hist_any_ref, od_ref, dem_ref,
                   hist_out_ref, spat_scr, slices_scr, rsems):
    i = pl.program_id(0)
    d = day_ref[0]
    hh = hour_ref[0]
    flat = d * NH + hh
    hour_len = jnp.maximum(6, hh - TIME_SLOT + 1)
    idx = ([(d - k, hh + 1) for k in range(TIME_SLOT)]
           + [(d - k, hh) for k in range(TIME_SLOT)]
           + [(d - k, hh + 2) for k in range(TIME_SLOT)]
           + [(d, hour_len + j) for j in range(TIME_SLOT)])

    @pl.when(i == 0)
    def _spatial():
        for t, (dd, th) in enumerate(idx):
            pltpu.make_async_copy(hist_any_ref.at[dd * NH + th],
                                  slices_scr.at[t], rsems.at[t]).start()

        h = jnp.dot(feat_ref[...], W_ref[...],
                    preferred_element_type=jnp.float32)

        def attn_agg(mask, a_ref):
            hl = jnp.dot(h, a_ref[:, :EMB].T,
                         preferred_element_type=jnp.float32)
            hr = jnp.dot(h, a_ref[:, EMB:].T,
                         preferred_element_type=jnp.float32)
            s = hl + hr.T
            s = jnp.where(s > 0, s, 0.2 * s)
            s = jnp.where(mask, s, -1e9)
            m = jnp.max(s, axis=1, keepdims=True)
            e = jnp.exp(s - m)
            att = e / jnp.sum(e, axis=1, keepdims=True)
            has_nbr = jnp.sum(mask.astype(jnp.float32), axis=1,
                              keepdims=True) > 0
            att = jnp.where(has_nbr, att, 0.0)
            return jnp.dot(att, h, preferred_element_type=jnp.float32)

        fo = fo_ref[...]
        row = jax.lax.broadcasted_iota(jnp.int32, (M, M), 0)
        col = jax.lax.broadcasted_iota(jnp.int32, (M, M), 1)
        spat_scr[:, :EMB] = h
        spat_scr[:, EMB:2 * EMB] = attn_agg(fo > 0.0, af_ref)
        spat_scr[:, 2 * EMB:3 * EMB] = attn_agg(fo.T > 0.0, ab_ref)
        spat_scr[:, 3 * EMB:] = attn_agg(
            (graph_ref[...] <= GEO_THR) & (row != col), ag_ref)

    hist_out_ref[...] = hist_blk_ref[...]

    @pl.when((flat >= i * C) & (flat < (i + 1) * C))
    def _scatter():
        hist_out_ref[flat - i * C] = spat_scr[...]

    @pl.when(i == G - 1)
    def _temporal():
        q = jnp.dot(feat1_ref[...], Wt_ref[...],
                    preferred_element_type=jnp.float32)
        spat = spat_scr[...]
        sels = []
        for t, (dd, th) in enumerate(idx):
            pltpu.make_async_copy(hist_any_ref.at[dd * NH + th],
                                  slices_scr.at[t], rsems.at[t]).wait()
            upd = (dd == d) & (th == hh)
            sels.append(jnp.where(upd, spat, slices_scr[t]))
        cols = [jnp.sum(s * q, axis=1, keepdims=True) for s in sels]
        scores = jnp.concatenate(cols, axis=1) / jnp.sqrt(jnp.float32(4 * EMB))
        m = jnp.max(scores, axis=1, keepdims=True)
        e = jnp.exp(scores - m)
        alpha = e / jnp.sum(e, axis=1, keepdims=True)
        temporal = alpha[:, 0:1] * sels[0]
        for t in range(1, T):
            temporal = temporal + alpha[:, t:t + 1] * sels[t]

        emb_o = jnp.dot(temporal, Po_ref[...],
                        preferred_element_type=jnp.float32)
        emb_d = jnp.dot(temporal, Pd_ref[...],
                        preferred_element_type=jnp.float32)
        t1 = jnp.dot(emb_o, tr_ref[...], preferred_element_type=jnp.float32)
        od = jax.lax.dot_general(t1, emb_d, (((1,), (1,)), ((), ())),
                                 preferred_element_type=jnp.float32)
        od = jnp.maximum(od, 0.0)
        od_ref[...] = od
        dem_ref[...] = jnp.sum(od, axis=1, keepdims=True) / jnp.float32(M)


def kernel(features, features_1, feat_out, history_spatial_embedding, day, hour,
           graph, W, a_f, a_b, a_g, W_t, P_o, P_d, tran_Matrix):
    hist = history_spatial_embedding
    hist3 = hist.reshape(G * C, M, 4 * EMB)
    day_arr = jnp.asarray(day, jnp.int32).reshape(1)
    hour_arr = jnp.asarray(hour, jnp.int32).reshape(1)
    vmem = pl.BlockSpec(memory_space=pltpu.MemorySpace.VMEM)
    smem = pl.BlockSpec(memory_space=pltpu.MemorySpace.SMEM)
    any_ = pl.BlockSpec(memory_space=pl.ANY)
    out = pl.pallas_call(
        _gallat_kernel,
        grid=(G,),
        out_shape=(
            jax.ShapeDtypeStruct((M, M), jnp.float32),
            jax.ShapeDtypeStruct((M, 1), jnp.float32),
            jax.ShapeDtypeStruct(hist3.shape, hist3.dtype),
        ),
        in_specs=[smem, smem] + [vmem] * 12
                 + [pl.BlockSpec((C, M, 4 * EMB), lambda i: (i, 0, 0)), any_],
        out_specs=(pl.BlockSpec((M, M), lambda i: (0, 0)),
                   pl.BlockSpec((M, 1), lambda i: (0, 0)),
                   pl.BlockSpec((C, M, 4 * EMB), lambda i: (i, 0, 0))),
        scratch_shapes=[
            pltpu.MemorySpace.VMEM((M, 4 * EMB), jnp.float32),
            pltpu.MemorySpace.VMEM((T, M, 4 * EMB), jnp.float32),
            pltpu.SemaphoreType.DMA((T,)),
        ],
    )(day_arr, hour_arr, features, features_1, feat_out, graph,
      W, a_f.reshape(1, 2 * EMB), a_b.reshape(1, 2 * EMB),
      a_g.reshape(1, 2 * EMB), W_t, P_o, P_d, tran_Matrix, hist3, hist3)
    return (out[0], out[1], out[2].reshape(hist.shape))


# speedup vs baseline: 1.0523x; 1.0523x over previous
import jax
import jax.numpy as jnp
from jax.experimental import pallas as pl
from jax.experimental.pallas import tpu as pltpu

M = 268
FEAT = 128
EMB = 64
TIME_SLOT = 4
GEO_THR = 3.0
T = 4 * TIME_SLOT
NH = 33
G = 10
C = 330 // G


def _gallat_kernel(day_ref, hour_ref, feat_ref, feat1_ref, fo_ref, graph_ref,
                   W_ref, af_ref, ab_ref, ag_ref, Wt_ref, Po_ref, Pd_ref,
                   tr_ref, hist_blk_ref, hist_any_ref, od_ref, dem_ref,
                   hist_out_ref, spat_scr, slices_scr, alpha_scr, temp_scr,
                   rsems):
    i = pl.program_id(0)
    d = day_ref[0]
    hh = hour_ref[0]
    flat = d * NH + hh
    hour_len = jnp.maximum(6, hh - TIME_SLOT + 1)
    idx = ([(d - k, hh + 1) for k in range(TIME_SLOT)]
           + [(d - k, hh) for k in range(TIME_SLOT)]
           + [(d - k, hh + 2) for k in range(TIME_SLOT)]
           + [(d, hour_len + j) for j in range(TIME_SLOT)])

    def attn_agg(mask, a_ref):
        h = spat_scr[:, :EMB]
        hl = jnp.dot(h, a_ref[:, :EMB].T, preferred_element_type=jnp.float32)
        hr = jnp.dot(h, a_ref[:, EMB:].T, preferred_element_type=jnp.float32)
        s = hl + hr.T
        s = jnp.where(s > 0, s, 0.2 * s)
        s = jnp.where(mask, s, -1e9)
        m = jnp.max(s, axis=1, keepdims=True)
        e = jnp.exp(s - m)
        att = e / jnp.sum(e, axis=1, keepdims=True)
        has_nbr = jnp.sum(mask.astype(jnp.float32), axis=1, keepdims=True) > 0
        att = jnp.where(has_nbr, att, 0.0)
        return jnp.dot(att, h, preferred_element_type=jnp.float32)

    @pl.when(i == 0)
    def _step0():
        for t, (dd, th) in enumerate(idx):
            pltpu.make_async_copy(hist_any_ref.at[dd * NH + th],
                                  slices_scr.at[t], rsems.at[t]).start()
        spat_scr[:, :EMB] = jnp.dot(feat_ref[...], W_ref[...],
                                    preferred_element_type=jnp.float32)

    @pl.when(i == 1)
    def _step1():
        spat_scr[:, EMB:2 * EMB] = attn_agg(fo_ref[...] > 0.0, af_ref)

    @pl.when(i == 2)
    def _step2():
        spat_scr[:, 2 * EMB:3 * EMB] = attn_agg(fo_ref[...].T > 0.0, ab_ref)

    @pl.when(i == 3)
    def _step3():
        row = jax.lax.broadcasted_iota(jnp.int32, (M, M), 0)
        col = jax.lax.broadcasted_iota(jnp.int32, (M, M), 1)
        geo = (graph_ref[...] <= GEO_THR) & (row != col)
        spat_scr[:, 3 * EMB:] = attn_agg(geo, ag_ref)

    @pl.when(i == 5)
    def _step5():
        spat = spat_scr[...]
        cols = []
        for t, (dd, th) in enumerate(idx):
            pltpu.make_async_copy(hist_any_ref.at[dd * NH + th],
                                  slices_scr.at[t], rsems.at[t]).wait()
            upd = (dd == d) & (th == hh)

            @pl.when(upd)
            def _():
                slices_scr[t] = spat
        q = jnp.dot(feat1_ref[...], Wt_ref[...],
                    preferred_element_type=jnp.float32)
        for t in range(T):
            cols.append(jnp.sum(slices_scr[t] * q, axis=1, keepdims=True))
        scores = jnp.concatenate(cols, axis=1) / jnp.sqrt(jnp.float32(4 * EMB))
        m = jnp.max(scores, axis=1, keepdims=True)
        e = jnp.exp(scores - m)
        alpha_scr[...] = e / jnp.sum(e, axis=1, keepdims=True)

    @pl.when(i == 6)
    def _step6():
        temporal = alpha_scr[:, 0:1] * slices_scr[0]
        for t in range(1, T):
            temporal = temporal + alpha_scr[:, t:t + 1] * slices_scr[t]
        temp_scr[...] = temporal

    @pl.when(i == 7)
    def _step7():
        temporal = temp_scr[...]
        emb_o = jnp.dot(temporal, Po_ref[...],
                        preferred_element_type=jnp.float32)
        emb_d = jnp.dot(temporal, Pd_ref[...],
                        preferred_element_type=jnp.float32)
        t1 = jnp.dot(emb_o, tr_ref[...], preferred_element_type=jnp.float32)
        od = jax.lax.dot_general(t1, emb_d, (((1,), (1,)), ((), ())),
                                 preferred_element_type=jnp.float32)
        od = jnp.maximum(od, 0.0)
        od_ref[...] = od
        dem_ref[...] = jnp.sum(od, axis=1, keepdims=True) / jnp.float32(M)

    hist_out_ref[...] = hist_blk_ref[...]

    @pl.when((flat >= i * C) & (flat < (i + 1) * C))
    def _scatter():
        hist_out_ref[flat - i * C] = spat_scr[...]


def kernel(features, features_1, feat_out, history_spatial_embedding, day, hour,
           graph, W, a_f, a_b, a_g, W_t, P_o, P_d, tran_Matrix):
    hist = history_spatial_embedding
    hist3 = hist.reshape(G * C, M, 4 * EMB)
    day_arr = jnp.asarray(day, jnp.int32).reshape(1)
    hour_arr = jnp.asarray(hour, jnp.int32).reshape(1)
    vmem = pl.BlockSpec(memory_space=pltpu.MemorySpace.VMEM)
    smem = pl.BlockSpec(memory_space=pltpu.MemorySpace.SMEM)
    any_ = pl.BlockSpec(memory_space=pl.ANY)
    out = pl.pallas_call(
        _gallat_kernel,
        grid=(G,),
        out_shape=(
            jax.ShapeDtypeStruct((M, M), jnp.float32),
            jax.ShapeDtypeStruct((M, 1), jnp.float32),
            jax.ShapeDtypeStruct(hist3.shape, hist3.dtype),
        ),
        in_specs=[smem, smem] + [vmem] * 12
                 + [pl.BlockSpec((C, M, 4 * EMB), lambda i: (i, 0, 0)), any_],
        out_specs=(pl.BlockSpec((M, M), lambda i: (0, 0)),
                   pl.BlockSpec((M, 1), lambda i: (0, 0)),
                   pl.BlockSpec((C, M, 4 * EMB), lambda i: (i, 0, 0))),
        scratch_shapes=[
            pltpu.MemorySpace.VMEM((M, 4 * EMB), jnp.float32),
            pltpu.MemorySpace.VMEM((T, M, 4 * EMB), jnp.float32),
            pltpu.MemorySpace.VMEM((M, T), jnp.float32),
            pltpu.MemorySpace.VMEM((M, 4 * EMB), jnp.float32),
            pltpu.SemaphoreType.DMA((T,)),
        ],
    )(day_arr, hour_arr, features, features_1, feat_out, graph,
      W, a_f.reshape(1, 2 * EMB), a_b.reshape(1, 2 * EMB),
      a_g.reshape(1, 2 * EMB), W_t, P_o, P_d, tran_Matrix, hist3, hist3)
    return (out[0], out[1], out[2].reshape(hist.shape))
